# baseline (speedup 1.0000x reference)
import functools

import jax
import jax.numpy as jnp
from jax import lax
from jax.experimental import pallas as pl
from jax.experimental.pallas import tpu as pltpu
from jax.experimental.pallas import tpu_sc as plsc

R = 20
N = 50000
E = 800000
K = 64

NC = 2
NS = 16
HALF = N // NC

CH = 128
EPAD = 835584
PER_TILE = EPAD // NS
NCHUNK = PER_TILE // CH
NDEEP = 3
NTRIP = NCHUNK // NDEEP

ACC_ROWS = 25088
TRASH = HALF + 8
ZROWS = ACC_ROWS // NS

CNT = N * R
CACC = 524288
CTRASH = HALF * R
CZ = CACC // NS
COPY_A = 31248
COPY_A_LAST = HALF * R - 15 * COPY_A

PW = EPAD // (NC * NS)
WCHUNKS = PW // CH

COPY_E = 1568
COPY_E_LAST = HALF - 15 * COPY_E

_mesh = functools.partial(
    plsc.VectorSubcoreMesh, core_axis_name="c", subcore_axis_name="s",
    num_cores=NC, num_subcores=NS)

_SC_PARAMS = pltpu.CompilerParams(use_tc_tiling_on_sc=False)


_GDN = lax.GatherDimensionNumbers(
    offset_dims=(), collapsed_slice_dims=(0,), start_index_map=(0,))


def _splat16(vec, t):
    idx = jnp.full((16, 1), t, jnp.int32)
    return lax.gather(vec, idx, _GDN, (1,),
                      mode=lax.GatherScatterMode.PROMISE_IN_BOUNDS)

def _count_body(didx_hbm, zeros_hbm, cnt_hbm, acc, dbuf, libuf, ones, vstage,
                semc0, semc1):
    c = lax.axis_index("c")
    s = lax.axis_index("s")
    pltpu.sync_copy(zeros_hbm, acc.at[pl.ds(s * CZ, CZ)])
    for g in range(CH // 16):
        ones[pl.ds(g * 16, 16)] = jnp.ones((16,), jnp.float32)
    plsc.subcore_barrier()

    coff = c * CTRASH
    tbase = s * PER_TILE

    def _calc(p):
        for g in range(CH // 16):
            v = dbuf[p, pl.ds(g * 16, 16)] - coff
            ok = (v >= 0) & (v < CTRASH)
            libuf[p, pl.ds(g * 16, 16)] = jnp.where(ok, v, CTRASH)

    semc = (semc0, semc1)
    for p in range(2):
        pltpu.sync_copy(didx_hbm.at[pl.ds(tbase + p * CH, CH)], dbuf.at[p])
        _calc(p)
        pltpu.async_copy(ones, acc.at[libuf.at[p]], semc[p], add=True)

    @pl.loop(1, NCHUNK // 2)
    def _pairs(g2):
        for p in range(2):
            i = 2 * g2 + p
            base = tbase + i * CH
            pltpu.sync_copy(didx_hbm.at[pl.ds(base, CH)], dbuf.at[p])
            pltpu.make_async_copy(ones, acc.at[libuf.at[p]], semc[p]).wait()
            _calc(p)
            pltpu.async_copy(ones, acc.at[libuf.at[p]], semc[p], add=True)

    for p in range(2):
        pltpu.make_async_copy(ones, acc.at[libuf.at[p]], semc[p]).wait()

    plsc.subcore_barrier()
    HC = COPY_A // 2
    HCL = COPY_A_LAST // 2

    @pl.when(s < NS - 1)
    def _():
        for k2 in range(2):
            o = s * COPY_A + k2 * HC
            pltpu.sync_copy(acc.at[pl.ds(o, HC)], vstage.at[pl.ds(0, HC)])
            pltpu.sync_copy(vstage.at[pl.ds(0, HC)],
                            cnt_hbm.at[pl.ds(c * CTRASH + o, HC)])

    @pl.when(s == NS - 1)
    def _():
        for k2 in range(2):
            o = 15 * COPY_A + k2 * HCL
            pltpu.sync_copy(acc.at[pl.ds(o, HCL)], vstage)
            pltpu.sync_copy(vstage, cnt_hbm.at[pl.ds(c * CTRASH + o, HCL)])


def _weight_body(didx_hbm, cnt_hbm, w_hbm, dibuf, cbuf, wbuf, semg0, semg1):
    c = lax.axis_index("c")
    s = lax.axis_index("s")
    wid = s * NC + c
    base_w = wid * PW
    semg = (semg0, semg1)

    def _gather(p, i):
        pltpu.async_copy(cnt_hbm.at[dibuf.at[p]], cbuf.at[p], semg[p])

    for p in range(2):
        pltpu.sync_copy(didx_hbm.at[pl.ds(base_w + p * CH, CH)], dibuf.at[p])
        _gather(p, p)

    @pl.loop(0, WCHUNKS // 2)
    def _pairs(g2):
        for p in range(2):
            i = 2 * g2 + p
            pltpu.make_async_copy(cnt_hbm.at[dibuf.at[p]], cbuf.at[p],
                                  semg[p]).wait()
            for g in range(CH // 16):
                cv = cbuf[p, pl.ds(g * 16, 16)]
                wbuf[pl.ds(g * 16, 16)] = 1.0 / jnp.maximum(cv, 1.0)
            pltpu.sync_copy(wbuf, w_hbm.at[pl.ds(base_w + i * CH, CH)])
            nxt = jnp.minimum(i + 2, WCHUNKS - 1)
            pltpu.sync_copy(didx_hbm.at[pl.ds(base_w + nxt * CH, CH)],
                            dibuf.at[p])
            _gather(p, nxt)

    for p in range(2):
        pltpu.make_async_copy(cnt_hbm.at[dibuf.at[p]], cbuf.at[p],
                              semg[p]).wait()


def _agg_body(h_tab_hbm, gidx_hbm, dst_hbm, w_hbm, zeros_hbm, agg_hbm,
              acc, gbuf, libuf, dbuf, wbuf, rows0, rows1, rows2, vstage,
              semg0, semg1, semg2, sems0, sems1, sems2, seml0, seml1, seml2):
    c = lax.axis_index("c")
    s = lax.axis_index("s")
    pltpu.sync_copy(zeros_hbm, acc.at[pl.ds(s * ZROWS, ZROWS)])
    plsc.subcore_barrier()

    doff = c * HALF
    rows = (rows0, rows1, rows2)
    semg = (semg0, semg1, semg2)
    sems = (sems0, sems1, sems2)
    seml = (seml0, seml1, seml2)
    tbase = s * PER_TILE

    def _load_linear(p, i):
        base = tbase + i * CH
        return (
            pltpu.async_copy(gidx_hbm.at[pl.ds(base, CH)], gbuf.at[p], seml[p]),
            pltpu.async_copy(dst_hbm.at[pl.ds(base, CH)], dbuf.at[p], seml[p]),
            pltpu.async_copy(w_hbm.at[pl.ds(base, CH)], wbuf.at[p], seml[p]),
        )

    def _calc_lidx(p):
        for g in range(CH // 16):
            v = dbuf[p, pl.ds(g * 16, 16)] - doff
            ok = (v >= 0) & (v < HALF)
            libuf[p, pl.ds(g * 16, 16)] = jnp.where(ok, v, TRASH)

    def _start_gather(p):
        return pltpu.async_copy(h_tab_hbm.at[gbuf.at[p]], rows[p], semg[p])

    def _scale(p):
        for b in range(CH // 16):
            wv = wbuf[p, pl.ds(b * 16, 16)]
            for t in range(16):
                j = b * 16 + t
                spl = _splat16(wv, t)
                for q in range(K // 16):
                    rows[p][j, pl.ds(q * 16, 16)] = (
                        rows[p][j, pl.ds(q * 16, 16)] * spl)

    for p in range(NDEEP):
        for ld in _load_linear(p, p):
            ld.wait()
        _calc_lidx(p)
        _start_gather(p)

    @pl.loop(0, NTRIP)
    def _trips(g3):
        i0 = NDEEP * g3
        for p in range(NDEEP):
            i = i0 + p
            pltpu.make_async_copy(h_tab_hbm.at[gbuf.at[p]], rows[p],
                                  semg[p]).wait()
            _scale(p)
            sc = pltpu.async_copy(rows[p], acc.at[libuf.at[p]], sems[p],
                                  add=True)
            nxt = jnp.minimum(i + NDEEP, NCHUNK - 1)
            lds = _load_linear(p, nxt)
            sc.wait()
            for ld in lds:
                ld.wait()
            _calc_lidx(p)
            _start_gather(p)

    for p in range(NDEEP):
        pltpu.make_async_copy(h_tab_hbm.at[gbuf.at[p]], rows[p],
                              semg[p]).wait()

    plsc.subcore_barrier()
    SR = 56

    @pl.when(s < NS - 1)
    def _():
        for k2 in range(28):
            o = s * COPY_E + k2 * SR
            pltpu.sync_copy(acc.at[pl.ds(o, SR)], vstage)
            pltpu.sync_copy(vstage, agg_hbm.at[pl.ds(c * HALF + o, SR)])

    @pl.when(s == NS - 1)
    def _():
        for k2 in range(26):
            o = 15 * COPY_E + k2 * SR
            pltpu.sync_copy(acc.at[pl.ds(o, SR)], vstage)
            pltpu.sync_copy(vstage, agg_hbm.at[pl.ds(c * HALF + o, SR)])
        o = 15 * COPY_E + 26 * SR
        rem = COPY_E_LAST - 26 * SR
        pltpu.sync_copy(acc.at[pl.ds(o, rem)], vstage.at[pl.ds(0, rem)])
        pltpu.sync_copy(vstage.at[pl.ds(0, rem)],
                        agg_hbm.at[pl.ds(c * HALF + o, rem)])


def _sc_counts(didx):
    zeros = jnp.zeros((CZ,), jnp.float32)
    return pl.kernel(
        _count_body,
        out_type=jax.ShapeDtypeStruct((CNT + 8,), jnp.float32),
        mesh=_mesh(),
        scratch_types=[
            pltpu.VMEM_SHARED((CACC,), jnp.float32),
            pltpu.VMEM((2, CH), jnp.int32),
            pltpu.VMEM((2, CH), jnp.int32),
            pltpu.VMEM((CH,), jnp.float32),
            pltpu.VMEM((COPY_A_LAST // 2,), jnp.float32),
            pltpu.SemaphoreType.DMA,
            pltpu.SemaphoreType.DMA,
        ],
        compiler_params=_SC_PARAMS,
    )(didx, zeros)


def _sc_weights(didx, cnt):
    return pl.kernel(
        _weight_body,
        out_type=jax.ShapeDtypeStruct((EPAD,), jnp.float32),
        mesh=_mesh(),
        scratch_types=[
            pltpu.VMEM((2, CH), jnp.int32),
            pltpu.VMEM((2, CH), jnp.float32),
            pltpu.VMEM((CH,), jnp.float32),
            pltpu.SemaphoreType.DMA,
            pltpu.SemaphoreType.DMA,
        ],
        compiler_params=_SC_PARAMS,
    )(didx, cnt)


def _sc_aggregate(h_tab, gidx, dst, w):
    zeros = jnp.zeros((ZROWS, K), jnp.float32)
    return pl.kernel(
        _agg_body,
        out_type=jax.ShapeDtypeStruct((N, K), jnp.float32),
        mesh=_mesh(),
        scratch_types=[
            pltpu.VMEM_SHARED((ACC_ROWS, K), jnp.float32),
            pltpu.VMEM((NDEEP, CH), jnp.int32),
            pltpu.VMEM((NDEEP, CH), jnp.int32),
            pltpu.VMEM((NDEEP, CH), jnp.int32),
            pltpu.VMEM((NDEEP, CH), jnp.float32),
            pltpu.VMEM((CH, K), jnp.float32),
            pltpu.VMEM((CH, K), jnp.float32),
            pltpu.VMEM((CH, K), jnp.float32),
            pltpu.VMEM((56, K), jnp.float32),
        ] + [pltpu.SemaphoreType.DMA] * 9,
        compiler_params=_SC_PARAMS,
    )(h_tab, gidx, dst, w, zeros)


TB_ROWS = 400
TNB = N // TB_ROWS
FB_ROWS = 2000
FNB = N // FB_ROWS


def _bn_of(sblk, st, g, be):
    mu = st[0:1, :] * (1.0 / N)
    var = st[1:2, :] * (1.0 / N) - mu * mu
    return (sblk - mu) * lax.rsqrt(var + 1e-5) * g + be


def _tab_body0(h_ref, w_ref, o_ref):
    h = h_ref[...]
    for r in range(R):
        o_ref[r, :, :] = jnp.dot(h, w_ref[r], preferred_element_type=jnp.float32)


def _tc_tables0(h, W):
    d = h.shape[1]
    out = pl.pallas_call(
        _tab_body0,
        grid=(TNB,),
        in_specs=[
            pl.BlockSpec((TB_ROWS, d), lambda i: (i, 0)),
            pl.BlockSpec((R, d, K), lambda i: (0, 0, 0)),
        ],
        out_specs=pl.BlockSpec((R, TB_ROWS, K), lambda i: (0, i, 0)),
        out_shape=jax.ShapeDtypeStruct((R, N, K), jnp.float32),
    )(h, W)
    return out.reshape(R * N, K)


def _tab_body_bn(s_ref, st_ref, g_ref, be_ref, w_ref, o_ref):
    h = _bn_of(s_ref[...], st_ref[...], g_ref[...], be_ref[...])
    for r in range(R):
        o_ref[r, :, :] = jnp.dot(h, w_ref[r], preferred_element_type=jnp.float32)


def _tc_tables_bn(s, st, g, be, W):
    out = pl.pallas_call(
        _tab_body_bn,
        grid=(TNB,),
        in_specs=[
            pl.BlockSpec((TB_ROWS, K), lambda i: (i, 0)),
            pl.BlockSpec((2, K), lambda i: (0, 0)),
            pl.BlockSpec((1, K), lambda i: (0, 0)),
            pl.BlockSpec((1, K), lambda i: (0, 0)),
            pl.BlockSpec((R, K, K), lambda i: (0, 0, 0)),
        ],
        out_specs=pl.BlockSpec((R, TB_ROWS, K), lambda i: (0, i, 0)),
        out_shape=jax.ShapeDtypeStruct((R, N, K), jnp.float32),
    )(s, st, g.reshape(1, K), be.reshape(1, K), W)
    return out.reshape(R * N, K)


def _stats_update(st_ref, sblk, nb):
    part = jnp.concatenate(
        [jnp.sum(sblk, 0, keepdims=True),
         jnp.sum(sblk * sblk, 0, keepdims=True)], axis=0)

    @pl.when(nb == 0)
    def _():
        st_ref[...] = part

    @pl.when(nb != 0)
    def _():
        st_ref[...] = st_ref[...] + part


def _base_body0(h_ref, root_ref, b_ref, agg_ref, s_ref, st_ref):
    sblk = (jnp.dot(h_ref[...], root_ref[...], preferred_element_type=jnp.float32)
            + b_ref[...] + agg_ref[...])
    s_ref[...] = sblk
    _stats_update(st_ref, sblk, pl.program_id(0))


def _tc_base0(h, root, b, agg):
    d = h.shape[1]
    return pl.pallas_call(
        _base_body0,
        grid=(FNB,),
        in_specs=[
            pl.BlockSpec((FB_ROWS, d), lambda i: (i, 0)),
            pl.BlockSpec((d, K), lambda i: (0, 0)),
            pl.BlockSpec((1, K), lambda i: (0, 0)),
            pl.BlockSpec((FB_ROWS, K), lambda i: (i, 0)),
        ],
        out_specs=[
            pl.BlockSpec((FB_ROWS, K), lambda i: (i, 0)),
            pl.BlockSpec((2, K), lambda i: (0, 0)),
        ],
        out_shape=[
            jax.ShapeDtypeStruct((N, K), jnp.float32),
            jax.ShapeDtypeStruct((2, K), jnp.float32),
        ],
    )(h, root, b.reshape(1, K), agg)


def _base_body_bn(sp_ref, stp_ref, g_ref, be_ref, root_ref, b_ref, agg_ref,
                  s_ref, st_ref):
    h = _bn_of(sp_ref[...], stp_ref[...], g_ref[...], be_ref[...])
    sblk = (jnp.dot(h, root_ref[...], preferred_element_type=jnp.float32)
            + b_ref[...] + agg_ref[...])
    s_ref[...] = sblk
    _stats_update(st_ref, sblk, pl.program_id(0))


def _tc_base_bn(sp, stp, g, be, root, b, agg):
    return pl.pallas_call(
        _base_body_bn,
        grid=(FNB,),
        in_specs=[
            pl.BlockSpec((FB_ROWS, K), lambda i: (i, 0)),
            pl.BlockSpec((2, K), lambda i: (0, 0)),
            pl.BlockSpec((1, K), lambda i: (0, 0)),
            pl.BlockSpec((1, K), lambda i: (0, 0)),
            pl.BlockSpec((K, K), lambda i: (0, 0)),
            pl.BlockSpec((1, K), lambda i: (0, 0)),
            pl.BlockSpec((FB_ROWS, K), lambda i: (i, 0)),
        ],
        out_specs=[
            pl.BlockSpec((FB_ROWS, K), lambda i: (i, 0)),
            pl.BlockSpec((2, K), lambda i: (0, 0)),
        ],
        out_shape=[
            jax.ShapeDtypeStruct((N, K), jnp.float32),
            jax.ShapeDtypeStruct((2, K), jnp.float32),
        ],
    )(sp, stp, g.reshape(1, K), be.reshape(1, K), root, b.reshape(1, K), agg)


def _bn_body(s_ref, st_ref, g_ref, be_ref, o_ref, *, relu):
    y = _bn_of(s_ref[...], st_ref[...], g_ref[...], be_ref[...])
    if relu:
        y = jnp.maximum(y, 0.0)
    o_ref[...] = y


def _tc_bn(s, st, g, be, relu):
    return pl.pallas_call(
        functools.partial(_bn_body, relu=relu),
        grid=(FNB,),
        in_specs=[
            pl.BlockSpec((FB_ROWS, K), lambda i: (i, 0)),
            pl.BlockSpec((2, K), lambda i: (0, 0)),
            pl.BlockSpec((1, K), lambda i: (0, 0)),
            pl.BlockSpec((1, K), lambda i: (0, 0)),
        ],
        out_specs=pl.BlockSpec((FB_ROWS, K), lambda i: (i, 0)),
        out_shape=jax.ShapeDtypeStruct((N, K), jnp.float32),
    )(s, st, g.reshape(1, K), be.reshape(1, K))


def kernel(x, W0, root0, b0, Wr, rootr, br, gammas, betas, edge_index,
           edge_attr, batch):
    del batch
    src = edge_index[0]
    dst = edge_index[1]
    et = edge_attr

    pad = EPAD - E
    gidx = jnp.concatenate([et * N + src, jnp.zeros((pad,), jnp.int32)])
    didx = jnp.concatenate([dst * R + et, jnp.full((pad,), CNT, jnp.int32)])
    dstp = jnp.concatenate([dst, jnp.full((pad,), -1, jnp.int32)])

    cnt = _sc_counts(didx)
    w = _sc_weights(didx, cnt)

    s = st = None
    for layer in range(4):
        if layer == 0:
            Wl, rootl, bl = W0, root0, b0
            tab = _tc_tables0(x, Wl)
        else:
            Wl, rootl, bl = Wr[layer - 1], rootr[layer - 1], br[layer - 1]
            tab = _tc_tables_bn(s, st, gammas[layer - 1], betas[layer - 1], Wl)
        agg = _sc_aggregate(tab, gidx, dstp, w)
        if layer == 0:
            s, st = _tc_base0(x, rootl, bl, agg)
        else:
            s, st = _tc_base_bn(s, st, gammas[layer - 1], betas[layer - 1],
                                rootl, bl, agg)
    return _tc_bn(s, st, gammas[3], betas[3], relu=True)


# speedup vs baseline: 1.7719x; 1.7719x over previous
import functools

import jax
import jax.numpy as jnp
from jax import lax
from jax.experimental import pallas as pl
from jax.experimental.pallas import tpu as pltpu
from jax.experimental.pallas import tpu_sc as plsc

R = 20
N = 50000
E = 800000
K = 64

NC = 2
NS = 16
HALF = N // NC

CH = 128
EPAD = 802816
PER_TILE = EPAD // NS
NCHUNK = PER_TILE // CH

ACC_ROWS = 25088
TRASH = HALF + 8
ZROWS = ACC_ROWS // NS

CNT = N * R
CACC = 524288
CTRASH = HALF * R
CZ = CACC // NS
COPY_A = 31248
COPY_A_LAST = HALF * R - 15 * COPY_A

PW = EPAD // (NC * NS)
WBLK = 512
NWBLK = PW // WBLK

COPY_E = 1568
COPY_E_LAST = HALF - 15 * COPY_E

_mesh = functools.partial(
    plsc.VectorSubcoreMesh, core_axis_name="c", subcore_axis_name="s",
    num_cores=NC, num_subcores=NS)

_SC_PARAMS = pltpu.CompilerParams(use_tc_tiling_on_sc=False)


_GDN = lax.GatherDimensionNumbers(
    offset_dims=(), collapsed_slice_dims=(0,), start_index_map=(0,))


def _splat16(vec, t):
    idx = jnp.full((16, 1), t, jnp.int32)
    return lax.gather(vec, idx, _GDN, (1,),
                      mode=lax.GatherScatterMode.PROMISE_IN_BOUNDS)

def _count_body(didx_hbm, zeros_hbm, cnt_hbm, acc, dbuf, libuf, ones, vstage):
    c = lax.axis_index("c")
    s = lax.axis_index("s")
    pltpu.sync_copy(zeros_hbm, acc.at[pl.ds(s * CZ, CZ)])
    for g in range(CH // 16):
        ones[pl.ds(g * 16, 16)] = jnp.ones((16,), jnp.float32)
    plsc.subcore_barrier()

    coff = c * CTRASH

    @pl.loop(0, NCHUNK)
    def _chunks(i):
        base = s * PER_TILE + i * CH
        pltpu.sync_copy(didx_hbm.at[pl.ds(base, CH)], dbuf)
        for g in range(CH // 16):
            v = dbuf[pl.ds(g * 16, 16)] - coff
            ok = (v >= 0) & (v < CTRASH)
            libuf[pl.ds(g * 16, 16)] = jnp.where(ok, v, CTRASH)
        pltpu.sync_copy(ones, acc.at[libuf], add=True)

    plsc.subcore_barrier()
    HC = COPY_A // 2
    HCL = COPY_A_LAST // 2

    @pl.when(s < NS - 1)
    def _():
        for k2 in range(2):
            o = s * COPY_A + k2 * HC
            pltpu.sync_copy(acc.at[pl.ds(o, HC)], vstage.at[pl.ds(0, HC)])
            pltpu.sync_copy(vstage.at[pl.ds(0, HC)],
                            cnt_hbm.at[pl.ds(c * CTRASH + o, HC)])

    @pl.when(s == NS - 1)
    def _():
        for k2 in range(2):
            o = 15 * COPY_A + k2 * HCL
            pltpu.sync_copy(acc.at[pl.ds(o, HCL)], vstage)
            pltpu.sync_copy(vstage, cnt_hbm.at[pl.ds(c * CTRASH + o, HCL)])


def _weight_body(didx_hbm, cnt_hbm, w_hbm, dibuf, cbuf, wbuf, sem):
    c = lax.axis_index("c")
    s = lax.axis_index("s")
    wid = s * NC + c
    base_w = wid * PW

    @pl.loop(0, NWBLK)
    def _blocks(i):
        base = base_w + i * WBLK
        for k in range(WBLK // CH):
            pltpu.sync_copy(didx_hbm.at[pl.ds(base + k * CH, CH)], dibuf.at[k])
            pltpu.async_copy(cnt_hbm.at[dibuf.at[k]], cbuf, sem).wait()
            for g in range(CH // 16):
                cv = cbuf[pl.ds(g * 16, 16)]
                wbuf[pl.ds(k * CH + g * 16, 16)] = 1.0 / jnp.maximum(cv, 1.0)
        pltpu.sync_copy(wbuf, w_hbm.at[pl.ds(base, WBLK)])


def _agg_body(h_tab_hbm, gidx_hbm, dst_hbm, w_hbm, zeros_hbm, agg_hbm,
              acc, gbuf, libuf, dbuf, wbuf, rows0, rows1, vstage,
              semg0, semg1, sems0, sems1, seml0, seml1):
    c = lax.axis_index("c")
    s = lax.axis_index("s")
    pltpu.sync_copy(zeros_hbm, acc.at[pl.ds(s * ZROWS, ZROWS)])
    plsc.subcore_barrier()

    doff = c * HALF
    rows = (rows0, rows1)
    semg = (semg0, semg1)
    sems = (sems0, sems1)
    seml = (seml0, seml1)
    tbase = s * PER_TILE

    def _load_linear(p, i):
        base = tbase + i * CH
        return (
            pltpu.async_copy(gidx_hbm.at[pl.ds(base, CH)], gbuf.at[p], seml[p]),
            pltpu.async_copy(dst_hbm.at[pl.ds(base, CH)], dbuf.at[p], seml[p]),
            pltpu.async_copy(w_hbm.at[pl.ds(base, CH)], wbuf.at[p], seml[p]),
        )

    def _calc_lidx(p):
        for g in range(CH // 16):
            v = dbuf[p, pl.ds(g * 16, 16)] - doff
            ok = (v >= 0) & (v < HALF)
            libuf[p, pl.ds(g * 16, 16)] = jnp.where(ok, v, TRASH)

    def _start_gather(p):
        return pltpu.async_copy(h_tab_hbm.at[gbuf.at[p]], rows[p], semg[p])

    def _scale(p):
        for b in range(CH // 16):
            wv = wbuf[p, pl.ds(b * 16, 16)]
            for t in range(16):
                j = b * 16 + t
                spl = _splat16(wv, t)
                for q in range(K // 16):
                    rows[p][j, pl.ds(q * 16, 16)] = (
                        rows[p][j, pl.ds(q * 16, 16)] * spl)

    for p in range(2):
        for ld in _load_linear(p, p):
            ld.wait()
        _calc_lidx(p)
        _start_gather(p)

    @pl.loop(0, NCHUNK // 2)
    def _pairs(g2):
        i0 = 2 * g2
        for p in range(2):
            i = i0 + p
            pltpu.make_async_copy(h_tab_hbm.at[gbuf.at[p]], rows[p],
                                  semg[p]).wait()
            _scale(p)
            sc = pltpu.async_copy(rows[p], acc.at[libuf.at[p]], sems[p],
                                  add=True)
            nxt = jnp.minimum(i + 2, NCHUNK - 1)
            lds = _load_linear(p, nxt)
            sc.wait()
            for ld in lds:
                ld.wait()
            _calc_lidx(p)
            _start_gather(p)

    for p in range(2):
        pltpu.make_async_copy(h_tab_hbm.at[gbuf.at[p]], rows[p],
                              semg[p]).wait()

    plsc.subcore_barrier()
    SR = 56

    @pl.when(s < NS - 1)
    def _():
        for k2 in range(28):
            o = s * COPY_E + k2 * SR
            pltpu.sync_copy(acc.at[pl.ds(o, SR)], vstage)
            pltpu.sync_copy(vstage, agg_hbm.at[pl.ds(c * HALF + o, SR)])

    @pl.when(s == NS - 1)
    def _():
        for k2 in range(26):
            o = 15 * COPY_E + k2 * SR
            pltpu.sync_copy(acc.at[pl.ds(o, SR)], vstage)
            pltpu.sync_copy(vstage, agg_hbm.at[pl.ds(c * HALF + o, SR)])
        o = 15 * COPY_E + 26 * SR
        rem = COPY_E_LAST - 26 * SR
        pltpu.sync_copy(acc.at[pl.ds(o, rem)], vstage.at[pl.ds(0, rem)])
        pltpu.sync_copy(vstage.at[pl.ds(0, rem)],
                        agg_hbm.at[pl.ds(c * HALF + o, rem)])


def _sc_counts(didx):
    zeros = jnp.zeros((CZ,), jnp.float32)
    return pl.kernel(
        _count_body,
        out_type=jax.ShapeDtypeStruct((CNT + 8,), jnp.float32),
        mesh=_mesh(),
        scratch_types=[
            pltpu.VMEM_SHARED((CACC,), jnp.float32),
            pltpu.VMEM((CH,), jnp.int32),
            pltpu.VMEM((CH,), jnp.int32),
            pltpu.VMEM((CH,), jnp.float32),
            pltpu.VMEM((COPY_A_LAST // 2,), jnp.float32),
        ],
        compiler_params=_SC_PARAMS,
    )(didx, zeros)


def _sc_weights(didx, cnt):
    return pl.kernel(
        _weight_body,
        out_type=jax.ShapeDtypeStruct((EPAD,), jnp.float32),
        mesh=_mesh(),
        scratch_types=[
            pltpu.VMEM((WBLK // CH, CH), jnp.int32),
            pltpu.VMEM((CH,), jnp.float32),
            pltpu.VMEM((WBLK,), jnp.float32),
            pltpu.SemaphoreType.DMA,
        ],
        compiler_params=_SC_PARAMS,
    )(didx, cnt)


def _sc_aggregate(h_tab, gidx, dst, w):
    zeros = jnp.zeros((ZROWS, K), jnp.float32)
    return pl.kernel(
        _agg_body,
        out_type=jax.ShapeDtypeStruct((N, K), jnp.float32),
        mesh=_mesh(),
        scratch_types=[
            pltpu.VMEM_SHARED((ACC_ROWS, K), jnp.float32),
            pltpu.VMEM((2, CH), jnp.int32),
            pltpu.VMEM((2, CH), jnp.int32),
            pltpu.VMEM((2, CH), jnp.int32),
            pltpu.VMEM((2, CH), jnp.float32),
            pltpu.VMEM((CH, K), jnp.float32),
            pltpu.VMEM((CH, K), jnp.float32),
            pltpu.VMEM((56, K), jnp.float32),
        ] + [pltpu.SemaphoreType.DMA] * 6,
        compiler_params=_SC_PARAMS,
    )(h_tab, gidx, dst, w, zeros)


TB_ROWS = 400
TNB = N // TB_ROWS
FB_ROWS = 2000
FNB = N // FB_ROWS


def _bn_of(sblk, st, g, be):
    mu = st[0:1, :] * (1.0 / N)
    var = st[1:2, :] * (1.0 / N) - mu * mu
    return (sblk - mu) * lax.rsqrt(var + 1e-5) * g + be


def _tab_body0(h_ref, w_ref, o_ref):
    h = h_ref[...]
    for r in range(R):
        o_ref[r, :, :] = jnp.dot(h, w_ref[r], preferred_element_type=jnp.float32)


def _tc_tables0(h, W):
    d = h.shape[1]
    out = pl.pallas_call(
        _tab_body0,
        grid=(TNB,),
        in_specs=[
            pl.BlockSpec((TB_ROWS, d), lambda i: (i, 0)),
            pl.BlockSpec((R, d, K), lambda i: (0, 0, 0)),
        ],
        out_specs=pl.BlockSpec((R, TB_ROWS, K), lambda i: (0, i, 0)),
        out_shape=jax.ShapeDtypeStruct((R, N, K), jnp.float32),
    )(h, W)
    return out.reshape(R * N, K)


def _tab_body_bn(s_ref, st_ref, g_ref, be_ref, w_ref, o_ref):
    h = _bn_of(s_ref[...], st_ref[...], g_ref[...], be_ref[...])
    for r in range(R):
        o_ref[r, :, :] = jnp.dot(h, w_ref[r], preferred_element_type=jnp.float32)


def _tc_tables_bn(s, st, g, be, W):
    out = pl.pallas_call(
        _tab_body_bn,
        grid=(TNB,),
        in_specs=[
            pl.BlockSpec((TB_ROWS, K), lambda i: (i, 0)),
            pl.BlockSpec((2, K), lambda i: (0, 0)),
            pl.BlockSpec((1, K), lambda i: (0, 0)),
            pl.BlockSpec((1, K), lambda i: (0, 0)),
            pl.BlockSpec((R, K, K), lambda i: (0, 0, 0)),
        ],
        out_specs=pl.BlockSpec((R, TB_ROWS, K), lambda i: (0, i, 0)),
        out_shape=jax.ShapeDtypeStruct((R, N, K), jnp.float32),
    )(s, st, g.reshape(1, K), be.reshape(1, K), W)
    return out.reshape(R * N, K)


def _stats_update(st_ref, sblk, nb):
    part = jnp.concatenate(
        [jnp.sum(sblk, 0, keepdims=True),
         jnp.sum(sblk * sblk, 0, keepdims=True)], axis=0)

    @pl.when(nb == 0)
    def _():
        st_ref[...] = part

    @pl.when(nb != 0)
    def _():
        st_ref[...] = st_ref[...] + part


def _base_body0(h_ref, root_ref, b_ref, agg_ref, s_ref, st_ref):
    sblk = (jnp.dot(h_ref[...], root_ref[...], preferred_element_type=jnp.float32)
            + b_ref[...] + agg_ref[...])
    s_ref[...] = sblk
    _stats_update(st_ref, sblk, pl.program_id(0))


def _tc_base0(h, root, b, agg):
    d = h.shape[1]
    return pl.pallas_call(
        _base_body0,
        grid=(FNB,),
        in_specs=[
            pl.BlockSpec((FB_ROWS, d), lambda i: (i, 0)),
            pl.BlockSpec((d, K), lambda i: (0, 0)),
            pl.BlockSpec((1, K), lambda i: (0, 0)),
            pl.BlockSpec((FB_ROWS, K), lambda i: (i, 0)),
        ],
        out_specs=[
            pl.BlockSpec((FB_ROWS, K), lambda i: (i, 0)),
            pl.BlockSpec((2, K), lambda i: (0, 0)),
        ],
        out_shape=[
            jax.ShapeDtypeStruct((N, K), jnp.float32),
            jax.ShapeDtypeStruct((2, K), jnp.float32),
        ],
    )(h, root, b.reshape(1, K), agg)


def _base_body_bn(sp_ref, stp_ref, g_ref, be_ref, root_ref, b_ref, agg_ref,
                  s_ref, st_ref):
    h = _bn_of(sp_ref[...], stp_ref[...], g_ref[...], be_ref[...])
    sblk = (jnp.dot(h, root_ref[...], preferred_element_type=jnp.float32)
            + b_ref[...] + agg_ref[...])
    s_ref[...] = sblk
    _stats_update(st_ref, sblk, pl.program_id(0))


def _tc_base_bn(sp, stp, g, be, root, b, agg):
    return pl.pallas_call(
        _base_body_bn,
        grid=(FNB,),
        in_specs=[
            pl.BlockSpec((FB_ROWS, K), lambda i: (i, 0)),
            pl.BlockSpec((2, K), lambda i: (0, 0)),
            pl.BlockSpec((1, K), lambda i: (0, 0)),
            pl.BlockSpec((1, K), lambda i: (0, 0)),
            pl.BlockSpec((K, K), lambda i: (0, 0)),
            pl.BlockSpec((1, K), lambda i: (0, 0)),
            pl.BlockSpec((FB_ROWS, K), lambda i: (i, 0)),
        ],
        out_specs=[
            pl.BlockSpec((FB_ROWS, K), lambda i: (i, 0)),
            pl.BlockSpec((2, K), lambda i: (0, 0)),
        ],
        out_shape=[
            jax.ShapeDtypeStruct((N, K), jnp.float32),
            jax.ShapeDtypeStruct((2, K), jnp.float32),
        ],
    )(sp, stp, g.reshape(1, K), be.reshape(1, K), root, b.reshape(1, K), agg)


def _bn_body(s_ref, st_ref, g_ref, be_ref, o_ref, *, relu):
    y = _bn_of(s_ref[...], st_ref[...], g_ref[...], be_ref[...])
    if relu:
        y = jnp.maximum(y, 0.0)
    o_ref[...] = y


def _tc_bn(s, st, g, be, relu):
    return pl.pallas_call(
        functools.partial(_bn_body, relu=relu),
        grid=(FNB,),
        in_specs=[
            pl.BlockSpec((FB_ROWS, K), lambda i: (i, 0)),
            pl.BlockSpec((2, K), lambda i: (0, 0)),
            pl.BlockSpec((1, K), lambda i: (0, 0)),
            pl.BlockSpec((1, K), lambda i: (0, 0)),
        ],
        out_specs=pl.BlockSpec((FB_ROWS, K), lambda i: (i, 0)),
        out_shape=jax.ShapeDtypeStruct((N, K), jnp.float32),
    )(s, st, g.reshape(1, K), be.reshape(1, K))


def kernel(x, W0, root0, b0, Wr, rootr, br, gammas, betas, edge_index,
           edge_attr, batch):
    del batch
    src = edge_index[0]
    dst = edge_index[1]
    et = edge_attr

    pad = EPAD - E
    gidx = jnp.concatenate([et * N + src, jnp.zeros((pad,), jnp.int32)])
    didx = jnp.concatenate([dst * R + et, jnp.full((pad,), CNT, jnp.int32)])
    dstp = jnp.concatenate([dst, jnp.full((pad,), -1, jnp.int32)])

    cnt = _sc_counts(didx)
    w = _sc_weights(didx, cnt)

    s = st = None
    for layer in range(4):
        if layer == 0:
            Wl, rootl, bl = W0, root0, b0
            tab = _tc_tables0(x, Wl)
        else:
            Wl, rootl, bl = Wr[layer - 1], rootr[layer - 1], br[layer - 1]
            tab = _tc_tables_bn(s, st, gammas[layer - 1], betas[layer - 1], Wl)
        agg = _sc_aggregate(tab, gidx, dstp, w)
        if layer == 0:
            s, st = _tc_base0(x, rootl, bl, agg)
        else:
            s, st = _tc_base_bn(s, st, gammas[layer - 1], betas[layer - 1],
                                rootl, bl, agg)
    return _tc_bn(s, st, gammas[3], betas[3], relu=True)


# speedup vs baseline: 1.7877x; 1.0089x over previous
import functools

import jax
import jax.numpy as jnp
from jax import lax
from jax.experimental import pallas as pl
from jax.experimental.pallas import tpu as pltpu
from jax.experimental.pallas import tpu_sc as plsc

R = 20
N = 50000
E = 800000
K = 64

NC = 2
NS = 16
HALF = N // NC

CH = 128
EPAD = 802816
PER_TILE = EPAD // NS
NCHUNK = PER_TILE // CH

ACC_ROWS = 25088
TRASH = HALF + 8
ZROWS = ACC_ROWS // NS

CNT = N * R
CACC = 524288
CTRASH = HALF * R
CZ = CACC // NS
COPY_A = 31248
COPY_A_LAST = HALF * R - 15 * COPY_A

PW = EPAD // (NC * NS)
WBLK = 512
NWBLK = PW // WBLK

COPY_E = 1568
COPY_E_LAST = HALF - 15 * COPY_E

_mesh = functools.partial(
    plsc.VectorSubcoreMesh, core_axis_name="c", subcore_axis_name="s",
    num_cores=NC, num_subcores=NS)

_SC_PARAMS = pltpu.CompilerParams(use_tc_tiling_on_sc=False)


_GDN = lax.GatherDimensionNumbers(
    offset_dims=(), collapsed_slice_dims=(0,), start_index_map=(0,))


def _splat16(vec, t):
    idx = jnp.full((16, 1), t, jnp.int32)
    return lax.gather(vec, idx, _GDN, (1,),
                      mode=lax.GatherScatterMode.PROMISE_IN_BOUNDS)

def _count_body(didx_hbm, zeros_hbm, cnt_hbm, acc, dbuf, libuf, ones, vstage):
    c = lax.axis_index("c")
    s = lax.axis_index("s")
    pltpu.sync_copy(zeros_hbm, acc.at[pl.ds(s * CZ, CZ)])
    for g in range(CH // 16):
        ones[pl.ds(g * 16, 16)] = jnp.ones((16,), jnp.float32)
    plsc.subcore_barrier()

    coff = c * CTRASH

    @pl.loop(0, NCHUNK)
    def _chunks(i):
        base = s * PER_TILE + i * CH
        pltpu.sync_copy(didx_hbm.at[pl.ds(base, CH)], dbuf)
        for g in range(CH // 16):
            v = dbuf[pl.ds(g * 16, 16)] - coff
            ok = (v >= 0) & (v < CTRASH)
            libuf[pl.ds(g * 16, 16)] = jnp.where(ok, v, CTRASH)
        pltpu.sync_copy(ones, acc.at[libuf], add=True)

    plsc.subcore_barrier()
    HC = COPY_A // 2
    HCL = COPY_A_LAST // 2

    @pl.when(s < NS - 1)
    def _():
        for k2 in range(2):
            o = s * COPY_A + k2 * HC
            pltpu.sync_copy(acc.at[pl.ds(o, HC)], vstage.at[pl.ds(0, HC)])
            pltpu.sync_copy(vstage.at[pl.ds(0, HC)],
                            cnt_hbm.at[pl.ds(c * CTRASH + o, HC)])

    @pl.when(s == NS - 1)
    def _():
        for k2 in range(2):
            o = 15 * COPY_A + k2 * HCL
            pltpu.sync_copy(acc.at[pl.ds(o, HCL)], vstage)
            pltpu.sync_copy(vstage, cnt_hbm.at[pl.ds(c * CTRASH + o, HCL)])


def _weight_body(didx_hbm, cnt_hbm, w_hbm, dibuf, cbuf, wbuf, sem):
    c = lax.axis_index("c")
    s = lax.axis_index("s")
    wid = s * NC + c
    base_w = wid * PW

    @pl.loop(0, NWBLK)
    def _blocks(i):
        base = base_w + i * WBLK
        for k in range(WBLK // CH):
            pltpu.sync_copy(didx_hbm.at[pl.ds(base + k * CH, CH)], dibuf.at[k])
            pltpu.async_copy(cnt_hbm.at[dibuf.at[k]], cbuf, sem).wait()
            for g in range(CH // 16):
                cv = cbuf[pl.ds(g * 16, 16)]
                wbuf[pl.ds(k * CH + g * 16, 16)] = 1.0 / jnp.maximum(cv, 1.0)
        pltpu.sync_copy(wbuf, w_hbm.at[pl.ds(base, WBLK)])


def _agg_body(h_tab_hbm, gidx_hbm, dst_hbm, w_hbm, zeros_hbm, agg_hbm,
              acc, gbuf, libuf, dbuf, wbuf, rows0, rows1, vstage,
              semg0, semg1, sems0, sems1, seml0, seml1):
    c = lax.axis_index("c")
    s = lax.axis_index("s")
    pltpu.sync_copy(zeros_hbm, acc.at[pl.ds(s * ZROWS, ZROWS)])
    plsc.subcore_barrier()

    doff = c * HALF
    rows = (rows0, rows1)
    semg = (semg0, semg1)
    sems = (sems0, sems1)
    seml = (seml0, seml1)
    tbase = s * PER_TILE

    def _load_linear(p, i):
        base = tbase + i * CH
        return (
            pltpu.async_copy(gidx_hbm.at[pl.ds(base, CH)], gbuf.at[p], seml[p]),
            pltpu.async_copy(dst_hbm.at[pl.ds(base, CH)], dbuf.at[p], seml[p]),
            pltpu.async_copy(w_hbm.at[pl.ds(base, CH)], wbuf.at[p], seml[p]),
        )

    def _calc_lidx(p):
        for g in range(CH // 16):
            v = dbuf[p, pl.ds(g * 16, 16)] - doff
            ok = (v >= 0) & (v < HALF)
            libuf[p, pl.ds(g * 16, 16)] = jnp.where(ok, v, TRASH)

    def _start_gather(p):
        return pltpu.async_copy(h_tab_hbm.at[gbuf.at[p]], rows[p], semg[p])

    def _scale(p):
        for b in range(CH // 16):
            wv = wbuf[p, pl.ds(b * 16, 16)]
            for t in range(16):
                j = b * 16 + t
                spl = _splat16(wv, t)
                for q in range(K // 16):
                    rows[p][j, pl.ds(q * 16, 16)] = (
                        rows[p][j, pl.ds(q * 16, 16)] * spl)

    for p in range(2):
        for ld in _load_linear(p, p):
            ld.wait()
        _calc_lidx(p)
        _start_gather(p)

    @pl.loop(0, NCHUNK // 2)
    def _pairs(g2):
        i0 = 2 * g2
        for p in range(2):
            i = i0 + p
            pltpu.make_async_copy(h_tab_hbm.at[gbuf.at[p]], rows[p],
                                  semg[p]).wait()
            _scale(p)
            sc = pltpu.async_copy(rows[p], acc.at[libuf.at[p]], sems[p],
                                  add=True)
            nxt = jnp.minimum(i + 2, NCHUNK - 1)
            lds = _load_linear(p, nxt)
            sc.wait()
            for ld in lds:
                ld.wait()
            _calc_lidx(p)
            _start_gather(p)

    for p in range(2):
        pltpu.make_async_copy(h_tab_hbm.at[gbuf.at[p]], rows[p],
                              semg[p]).wait()

    plsc.subcore_barrier()
    SR = 56

    @pl.when(s < NS - 1)
    def _():
        for k2 in range(28):
            o = s * COPY_E + k2 * SR
            pltpu.sync_copy(acc.at[pl.ds(o, SR)], vstage)
            pltpu.sync_copy(vstage, agg_hbm.at[pl.ds(c * HALF + o, SR)])

    @pl.when(s == NS - 1)
    def _():
        for k2 in range(26):
            o = 15 * COPY_E + k2 * SR
            pltpu.sync_copy(acc.at[pl.ds(o, SR)], vstage)
            pltpu.sync_copy(vstage, agg_hbm.at[pl.ds(c * HALF + o, SR)])
        o = 15 * COPY_E + 26 * SR
        rem = COPY_E_LAST - 26 * SR
        pltpu.sync_copy(acc.at[pl.ds(o, rem)], vstage.at[pl.ds(0, rem)])
        pltpu.sync_copy(vstage.at[pl.ds(0, rem)],
                        agg_hbm.at[pl.ds(c * HALF + o, rem)])


def _sc_counts(didx):
    zeros = jnp.zeros((CZ,), jnp.float32)
    return pl.kernel(
        _count_body,
        out_type=jax.ShapeDtypeStruct((CNT + 8,), jnp.float32),
        mesh=_mesh(),
        scratch_types=[
            pltpu.VMEM_SHARED((CACC,), jnp.float32),
            pltpu.VMEM((CH,), jnp.int32),
            pltpu.VMEM((CH,), jnp.int32),
            pltpu.VMEM((CH,), jnp.float32),
            pltpu.VMEM((COPY_A_LAST // 2,), jnp.float32),
        ],
        compiler_params=_SC_PARAMS,
    )(didx, zeros)


def _sc_weights(didx, cnt):
    return pl.kernel(
        _weight_body,
        out_type=jax.ShapeDtypeStruct((EPAD,), jnp.float32),
        mesh=_mesh(),
        scratch_types=[
            pltpu.VMEM((WBLK // CH, CH), jnp.int32),
            pltpu.VMEM((CH,), jnp.float32),
            pltpu.VMEM((WBLK,), jnp.float32),
            pltpu.SemaphoreType.DMA,
        ],
        compiler_params=_SC_PARAMS,
    )(didx, cnt)


def _sc_aggregate(h_tab, gidx, dst, w):
    zeros = jnp.zeros((ZROWS, K), jnp.float32)
    return pl.kernel(
        _agg_body,
        out_type=jax.ShapeDtypeStruct((N, K), jnp.float32),
        mesh=_mesh(),
        scratch_types=[
            pltpu.VMEM_SHARED((ACC_ROWS, K), jnp.float32),
            pltpu.VMEM((2, CH), jnp.int32),
            pltpu.VMEM((2, CH), jnp.int32),
            pltpu.VMEM((2, CH), jnp.int32),
            pltpu.VMEM((2, CH), jnp.float32),
            pltpu.VMEM((CH, K), jnp.float32),
            pltpu.VMEM((CH, K), jnp.float32),
            pltpu.VMEM((56, K), jnp.float32),
        ] + [pltpu.SemaphoreType.DMA] * 6,
        compiler_params=_SC_PARAMS,
    )(h_tab, gidx, dst, w, zeros)


TB_ROWS = 1000
TNB = N // TB_ROWS
FB_ROWS = 2000
FNB = N // FB_ROWS


def _bn_of(sblk, st, g, be):
    mu = st[0:1, :] * (1.0 / N)
    var = st[1:2, :] * (1.0 / N) - mu * mu
    return (sblk - mu) * lax.rsqrt(var + 1e-5) * g + be


def _tab_body0(h_ref, w_ref, o_ref):
    h = h_ref[...]
    for r in range(R):
        o_ref[r, :, :] = jnp.dot(h, w_ref[r], preferred_element_type=jnp.float32)


def _tc_tables0(h, W):
    d = h.shape[1]
    out = pl.pallas_call(
        _tab_body0,
        grid=(TNB,),
        in_specs=[
            pl.BlockSpec((TB_ROWS, d), lambda i: (i, 0)),
            pl.BlockSpec((R, d, K), lambda i: (0, 0, 0)),
        ],
        out_specs=pl.BlockSpec((R, TB_ROWS, K), lambda i: (0, i, 0)),
        out_shape=jax.ShapeDtypeStruct((R, N, K), jnp.float32),
    )(h, W)
    return out.reshape(R * N, K)


def _tab_body_bn(s_ref, st_ref, g_ref, be_ref, w_ref, o_ref):
    h = _bn_of(s_ref[...], st_ref[...], g_ref[...], be_ref[...])
    for r in range(R):
        o_ref[r, :, :] = jnp.dot(h, w_ref[r], preferred_element_type=jnp.float32)


def _tc_tables_bn(s, st, g, be, W):
    out = pl.pallas_call(
        _tab_body_bn,
        grid=(TNB,),
        in_specs=[
            pl.BlockSpec((TB_ROWS, K), lambda i: (i, 0)),
            pl.BlockSpec((2, K), lambda i: (0, 0)),
            pl.BlockSpec((1, K), lambda i: (0, 0)),
            pl.BlockSpec((1, K), lambda i: (0, 0)),
            pl.BlockSpec((R, K, K), lambda i: (0, 0, 0)),
        ],
        out_specs=pl.BlockSpec((R, TB_ROWS, K), lambda i: (0, i, 0)),
        out_shape=jax.ShapeDtypeStruct((R, N, K), jnp.float32),
    )(s, st, g.reshape(1, K), be.reshape(1, K), W)
    return out.reshape(R * N, K)


def _stats_update(st_ref, sblk, nb):
    part = jnp.concatenate(
        [jnp.sum(sblk, 0, keepdims=True),
         jnp.sum(sblk * sblk, 0, keepdims=True)], axis=0)

    @pl.when(nb == 0)
    def _():
        st_ref[...] = part

    @pl.when(nb != 0)
    def _():
        st_ref[...] = st_ref[...] + part


def _base_body0(h_ref, root_ref, b_ref, agg_ref, s_ref, st_ref):
    sblk = (jnp.dot(h_ref[...], root_ref[...], preferred_element_type=jnp.float32)
            + b_ref[...] + agg_ref[...])
    s_ref[...] = sblk
    _stats_update(st_ref, sblk, pl.program_id(0))


def _tc_base0(h, root, b, agg):
    d = h.shape[1]
    return pl.pallas_call(
        _base_body0,
        grid=(FNB,),
        in_specs=[
            pl.BlockSpec((FB_ROWS, d), lambda i: (i, 0)),
            pl.BlockSpec((d, K), lambda i: (0, 0)),
            pl.BlockSpec((1, K), lambda i: (0, 0)),
            pl.BlockSpec((FB_ROWS, K), lambda i: (i, 0)),
        ],
        out_specs=[
            pl.BlockSpec((FB_ROWS, K), lambda i: (i, 0)),
            pl.BlockSpec((2, K), lambda i: (0, 0)),
        ],
        out_shape=[
            jax.ShapeDtypeStruct((N, K), jnp.float32),
            jax.ShapeDtypeStruct((2, K), jnp.float32),
        ],
    )(h, root, b.reshape(1, K), agg)


def _base_body_bn(sp_ref, stp_ref, g_ref, be_ref, root_ref, b_ref, agg_ref,
                  s_ref, st_ref):
    h = _bn_of(sp_ref[...], stp_ref[...], g_ref[...], be_ref[...])
    sblk = (jnp.dot(h, root_ref[...], preferred_element_type=jnp.float32)
            + b_ref[...] + agg_ref[...])
    s_ref[...] = sblk
    _stats_update(st_ref, sblk, pl.program_id(0))


def _tc_base_bn(sp, stp, g, be, root, b, agg):
    return pl.pallas_call(
        _base_body_bn,
        grid=(FNB,),
        in_specs=[
            pl.BlockSpec((FB_ROWS, K), lambda i: (i, 0)),
            pl.BlockSpec((2, K), lambda i: (0, 0)),
            pl.BlockSpec((1, K), lambda i: (0, 0)),
            pl.BlockSpec((1, K), lambda i: (0, 0)),
            pl.BlockSpec((K, K), lambda i: (0, 0)),
            pl.BlockSpec((1, K), lambda i: (0, 0)),
            pl.BlockSpec((FB_ROWS, K), lambda i: (i, 0)),
        ],
        out_specs=[
            pl.BlockSpec((FB_ROWS, K), lambda i: (i, 0)),
            pl.BlockSpec((2, K), lambda i: (0, 0)),
        ],
        out_shape=[
            jax.ShapeDtypeStruct((N, K), jnp.float32),
            jax.ShapeDtypeStruct((2, K), jnp.float32),
        ],
    )(sp, stp, g.reshape(1, K), be.reshape(1, K), root, b.reshape(1, K), agg)


def _bn_body(s_ref, st_ref, g_ref, be_ref, o_ref, *, relu):
    y = _bn_of(s_ref[...], st_ref[...], g_ref[...], be_ref[...])
    if relu:
        y = jnp.maximum(y, 0.0)
    o_ref[...] = y


def _tc_bn(s, st, g, be, relu):
    return pl.pallas_call(
        functools.partial(_bn_body, relu=relu),
        grid=(FNB,),
        in_specs=[
            pl.BlockSpec((FB_ROWS, K), lambda i: (i, 0)),
            pl.BlockSpec((2, K), lambda i: (0, 0)),
            pl.BlockSpec((1, K), lambda i: (0, 0)),
            pl.BlockSpec((1, K), lambda i: (0, 0)),
        ],
        out_specs=pl.BlockSpec((FB_ROWS, K), lambda i: (i, 0)),
        out_shape=jax.ShapeDtypeStruct((N, K), jnp.float32),
    )(s, st, g.reshape(1, K), be.reshape(1, K))


def kernel(x, W0, root0, b0, Wr, rootr, br, gammas, betas, edge_index,
           edge_attr, batch):
    del batch
    src = edge_index[0]
    dst = edge_index[1]
    et = edge_attr

    pad = EPAD - E
    gidx = jnp.concatenate([et * N + src, jnp.zeros((pad,), jnp.int32)])
    didx = jnp.concatenate([dst * R + et, jnp.full((pad,), CNT, jnp.int32)])
    dstp = jnp.concatenate([dst, jnp.full((pad,), -1, jnp.int32)])

    cnt = _sc_counts(didx)
    w = _sc_weights(didx, cnt)

    s = st = None
    for layer in range(4):
        if layer == 0:
            Wl, rootl, bl = W0, root0, b0
            tab = _tc_tables0(x, Wl)
        else:
            Wl, rootl, bl = Wr[layer - 1], rootr[layer - 1], br[layer - 1]
            tab = _tc_tables_bn(s, st, gammas[layer - 1], betas[layer - 1], Wl)
        agg = _sc_aggregate(tab, gidx, dstp, w)
        if layer == 0:
            s, st = _tc_base0(x, rootl, bl, agg)
        else:
            s, st = _tc_base_bn(s, st, gammas[layer - 1], betas[layer - 1],
                                rootl, bl, agg)
    return _tc_bn(s, st, gammas[3], betas[3], relu=True)
